# baseline (speedup 1.0000x reference)
import functools
import math

import jax
import jax.numpy as jnp
from jax import lax
from jax.experimental import pallas as pl
from jax.experimental.pallas import tpu as pltpu
from jax.experimental.pallas import tpu_sc as plsc

_info = plsc.get_sparse_core_info()
_NC, _NS, _L = _info.num_cores, _info.num_subcores, _info.num_lanes
_NW = _NC * _NS


@functools.lru_cache(maxsize=None)
def _build(N, S, D, V):
    b_per_w = N // _NW
    C = 32
    n_chunks = b_per_w // C
    scale = float(math.sqrt(D))
    mesh = plsc.VectorSubcoreMesh(core_axis_name="c", subcore_axis_name="s")

    @functools.partial(
        pl.kernel,
        out_type=jax.ShapeDtypeStruct((N, D), jnp.float32),
        mesh=mesh,
        scratch_types=[
            pltpu.VMEM((C,), jnp.int32),
            pltpu.VMEM((C, D), jnp.float32),
            pltpu.VMEM((C, D), jnp.float32),
            pltpu.SemaphoreType.DMA,
        ],
    )
    def emb(x_hbm, table_hbm, pe_hbm, out_hbm, idx_v, rows_v, pe_v, sem):
        wid = lax.axis_index("s") * _NC + lax.axis_index("c")
        base = wid * b_per_w
        pe_base = lax.rem(base, S)
        for k in range(n_chunks):
            off = base + k * C
            pltpu.sync_copy(x_hbm.at[pl.ds(off, C)], idx_v)
            pltpu.async_copy(table_hbm.at[idx_v], rows_v, sem).wait()
            pltpu.sync_copy(pe_hbm.at[pl.ds(pe_base + k * C, C)], pe_v)

            def row_body(r, _):
                def vec_body(j, _2):
                    sl = pl.ds(j * _L, _L)
                    rows_v[r, sl] = rows_v[r, sl] * scale + pe_v[r, sl]
                    return 0
                lax.fori_loop(0, D // _L, vec_body, 0)
                return 0

            lax.fori_loop(0, C, row_body, 0)
            pltpu.sync_copy(rows_v, out_hbm.at[pl.ds(off, C)])

    return emb


def kernel(x, table, pe):
    B, S = x.shape
    V, D = table.shape
    N = B * S
    xf = x.reshape(N).astype(jnp.int32)
    emb = _build(N, S, D, V)
    out = emb(xf, table, pe)
    return out.reshape(B, S, D)


# speedup vs baseline: 2.6226x; 2.6226x over previous
import functools
import math

import jax
import jax.numpy as jnp
from jax import lax
from jax.experimental import pallas as pl
from jax.experimental.pallas import tpu as pltpu
from jax.experimental.pallas import tpu_sc as plsc

_info = plsc.get_sparse_core_info()
_NC, _NS, _L = _info.num_cores, _info.num_subcores, _info.num_lanes
_NW = _NC * _NS


@functools.lru_cache(maxsize=None)
def _build(N, S, D, V):
    b_per_w = N // _NW
    C = 16
    NBUF = 2
    n_chunks = b_per_w // C
    nvec = D // _L
    assert nvec & (nvec - 1) == 0
    nvec_shift = nvec.bit_length() - 1
    scale = float(math.sqrt(D))
    mesh = plsc.VectorSubcoreMesh(core_axis_name="c", subcore_axis_name="s")

    @functools.partial(
        pl.kernel,
        out_type=jax.ShapeDtypeStruct((N, D), jnp.float32),
        mesh=mesh,
        scratch_types=[
            pltpu.VMEM((b_per_w,), jnp.int32),
            pltpu.VMEM((NBUF, C, D), jnp.float32),
            pltpu.VMEM((NBUF, C, D), jnp.float32),
            pltpu.VMEM((NBUF, C, D), jnp.float32),
            pltpu.SemaphoreType.DMA((NBUF,)),
            pltpu.SemaphoreType.DMA((NBUF,)),
            pltpu.SemaphoreType.DMA((NBUF,)),
        ],
    )
    def emb(x_hbm, table_hbm, pe_hbm, out_hbm, idx_all, rows, peb, outb,
            gsem, psem, ssem):
        wid = lax.axis_index("s") * _NC + lax.axis_index("c")
        base = wid * b_per_w
        pe_base = lax.rem(base, S)
        pltpu.sync_copy(x_hbm.at[pl.ds(base, b_per_w)], idx_all)

        gh, ph, sh = {}, {}, {}

        def start_in(k):
            b = k % NBUF
            gh[k] = pltpu.async_copy(
                table_hbm.at[idx_all.at[pl.ds(k * C, C)]], rows.at[b],
                gsem.at[b])
            ph[k] = pltpu.async_copy(
                pe_hbm.at[pl.ds(pe_base + k * C, C)], peb.at[b], psem.at[b])

        for k in range(NBUF):
            start_in(k)
        for k in range(n_chunks):
            b = k % NBUF
            gh[k].wait()
            ph[k].wait()
            if k >= NBUF:
                sh[k - NBUF].wait()

            @plsc.parallel_loop(0, C * nvec, unroll=8)
            def _(i):
                r = lax.shift_right_logical(i, nvec_shift)
                sl = pl.ds((i & (nvec - 1)) * _L, _L)
                outb[b, r, sl] = rows[b, r, sl] * scale + peb[b, r, sl]

            sh[k] = pltpu.async_copy(
                outb.at[b], out_hbm.at[pl.ds(base + k * C, C)], ssem.at[b])
            if k + NBUF < n_chunks:
                start_in(k + NBUF)
        for k in range(n_chunks - NBUF, n_chunks):
            sh[k].wait()

    return emb


def kernel(x, table, pe):
    B, S = x.shape
    V, D = table.shape
    N = B * S
    xf = x.reshape(N).astype(jnp.int32)
    emb = _build(N, S, D, V)
    out = emb(xf, table, pe)
    return out.reshape(B, S, D)
